# baseline (speedup 1.0000x reference)
import functools

import jax
import jax.numpy as jnp
from jax import lax
from jax.experimental import pallas as pl
from jax.experimental.pallas import tpu as pltpu
from jax.experimental.pallas import tpu_sc as plsc

NC = 2
NS = 16
NW = NC * NS

EMBED = 64
SEQ = 50
BATCH = 1024
SAMPLE = 20
NTOK = SEQ * BATCH
HALF = 512


def _sc_gather_paired(table, idx, W, K):
    N = idx.shape[0]
    D = table.shape[1]
    n_per_w = N // NW
    R = n_per_w // W
    n_chunks = R // K
    F = K * W
    assert N == NW * R * W and R == n_chunks * K
    assert HALF % F == 0 and N % 1024 == 0

    mesh = plsc.VectorSubcoreMesh(core_axis_name="c", subcore_axis_name="s")

    @functools.partial(
        pl.kernel,
        mesh=mesh,
        out_type=jax.ShapeDtypeStruct((N // 1024, HALF, 2, D), jnp.float32),
        compiler_params=pltpu.CompilerParams(use_tc_tiling_on_sc=False),
        scratch_types=[
            pltpu.VMEM((R, W), jnp.int32),
            pltpu.VMEM((F, D), jnp.float32),
            pltpu.SemaphoreType.DMA,
        ],
    )
    def gather_kernel(table_hbm, idx_hbm, out_hbm, idx_v, rows_v, sem):
        wid = lax.axis_index("s") * NC + lax.axis_index("c")
        base = wid * n_per_w
        pltpu.sync_copy(idx_hbm.at[wid], idx_v)

        def chunk_body(i, carry):
            cps = []
            for j in range(K):
                cps.append(
                    pltpu.async_copy(
                        table_hbm.at[idx_v.at[i * K + j]],
                        rows_v.at[pl.ds(j * W, W)],
                        sem,
                    )
                )
            for cp in cps:
                cp.wait()
            a = base + i * F
            c = a // 1024
            m = a % 1024
            h = m // HALF
            rr = m % HALF
            pltpu.sync_copy(rows_v, out_hbm.at[c, pl.ds(rr, F), h])
            return carry

        lax.fori_loop(0, n_chunks, chunk_body, 0)

    return gather_kernel(table, idx.reshape(NW, R, W))


def _sc_gather_paired_db(table, idx, W, K):
    N = idx.shape[0]
    D = table.shape[1]
    n_per_w = N // NW
    R = n_per_w // W
    n_chunks = R // K
    F = K * W
    assert N == NW * R * W and R == n_chunks * K and n_chunks % 2 == 0
    assert HALF % F == 0 and N % 1024 == 0
    n2 = n_chunks // 2

    mesh = plsc.VectorSubcoreMesh(core_axis_name="c", subcore_axis_name="s")

    @functools.partial(
        pl.kernel,
        mesh=mesh,
        out_type=jax.ShapeDtypeStruct((N // 1024, HALF, 2, D), jnp.float32),
        compiler_params=pltpu.CompilerParams(use_tc_tiling_on_sc=False),
        scratch_types=[
            pltpu.VMEM((R, W), jnp.int32),
            pltpu.VMEM((F, D), jnp.float32),
            pltpu.VMEM((F, D), jnp.float32),
            pltpu.SemaphoreType.DMA,
            pltpu.SemaphoreType.DMA,
            pltpu.SemaphoreType.DMA,
            pltpu.SemaphoreType.DMA,
        ],
    )
    def gather_kernel(table_hbm, idx_hbm, out_hbm, idx_v, rows0, rows1,
                      semg0, semg1, semf0, semf1):
        wid = lax.axis_index("s") * NC + lax.axis_index("c")
        base = wid * n_per_w
        pltpu.sync_copy(idx_hbm.at[wid], idx_v)

        def fire(i, buf, sem):
            for j in range(K):
                pltpu.async_copy(
                    table_hbm.at[idx_v.at[i * K + j]],
                    buf.at[pl.ds(j * W, W)],
                    sem,
                )

        def wait_gathers(buf, sem):
            for j in range(K):
                pltpu.make_async_copy(
                    table_hbm.at[idx_v.at[0]], buf.at[pl.ds(j * W, W)], sem
                ).wait()

        def dst(i):
            a = base + i * F
            return out_hbm.at[a // 1024, pl.ds((a % 1024) % HALF, F),
                              (a % 1024) // HALF]

        fire(0, rows0, semg0)

        def body(p, carry):
            i0 = p * 2
            wait_gathers(rows0, semg0)

            @pl.when(p > 0)
            def _():
                pltpu.make_async_copy(rows1, dst(i0 - 1), semf1).wait()

            fire(i0 + 1, rows1, semg1)
            pltpu.async_copy(rows0, dst(i0), semf0)
            wait_gathers(rows1, semg1)
            pltpu.make_async_copy(rows0, dst(i0), semf0).wait()

            @pl.when(p < n2 - 1)
            def _():
                fire(i0 + 2, rows0, semg0)

            pltpu.async_copy(rows1, dst(i0 + 1), semf1)
            return carry

        lax.fori_loop(0, n2, body, 0)
        pltpu.make_async_copy(rows1, dst(n_chunks - 1), semf1).wait()

    return gather_kernel(table, idx.reshape(NW, R, W))


_TB = 5120


def _transpose_body(x_ref, o_ref):
    x = x_ref[0]
    eye = jnp.eye(128, dtype=jnp.float32)
    dn = (((1,), (1,)), ((), ()))
    y = lax.dot_general(eye, x, dn,
                        precision=lax.Precision.HIGHEST,
                        preferred_element_type=jnp.float32)
    for g in range(_TB // 1024):
        o_ref[0, :, g * 1024 : g * 1024 + HALF] = y[0:EMBED, g * HALF : (g + 1) * HALF]
        o_ref[0, :, g * 1024 + HALF : (g + 1) * 1024] = y[EMBED:128, g * HALF : (g + 1) * HALF]


def _tc_transpose(x_paired, nj, T):
    x3 = x_paired.reshape(nj, T // 2, 128)
    n_c = T // _TB
    return pl.pallas_call(
        _transpose_body,
        grid=(nj, n_c),
        in_specs=[pl.BlockSpec((1, _TB // 2, 128), lambda j, c: (j, c, 0))],
        out_specs=pl.BlockSpec((1, EMBED, _TB), lambda j, c: (j, 0, c)),
        out_shape=jax.ShapeDtypeStruct((nj, EMBED, T), jnp.float32),
    )(x3)


def _tc_transpose_into(x_paired, prev, j0, nj_chunk, nj_total):
    x3 = x_paired.reshape(nj_chunk, NTOK // 2, 128)
    n_c = NTOK // _TB
    out_sds = jax.ShapeDtypeStruct((nj_total, EMBED, NTOK), jnp.float32)
    if prev is None:
        return pl.pallas_call(
            _transpose_body,
            grid=(nj_chunk, n_c),
            in_specs=[pl.BlockSpec((1, _TB // 2, 128), lambda j, c: (j, c, 0))],
            out_specs=pl.BlockSpec((1, EMBED, _TB), lambda j, c: (j + j0, 0, c)),
            out_shape=out_sds,
        )(x3)

    def body(x_ref, prev_ref, o_ref):
        _transpose_body(x_ref, o_ref)

    return pl.pallas_call(
        body,
        grid=(nj_chunk, n_c),
        in_specs=[
            pl.BlockSpec((1, _TB // 2, 128), lambda j, c: (j, c, 0)),
            pl.BlockSpec(memory_space=pl.ANY),
        ],
        out_specs=pl.BlockSpec((1, EMBED, _TB), lambda j, c: (j + j0, 0, c)),
        out_shape=out_sds,
        input_output_aliases={1: 0},
    )(x3, prev)


def _lstm_body(x_ref, wih_ref, whh_ref, b_ref, out_ref, ht_scr, c_scr):
    t = pl.program_id(0)

    @pl.when(t == 0)
    def _init():
        ht_scr[...] = jnp.zeros_like(ht_scr)
        c_scr[...] = jnp.zeros_like(c_scr)

    x2 = x_ref[...]
    xt = jnp.concatenate([x2[:, 0:EMBED], x2[:, EMBED:128]], axis=0)
    dn0 = (((0,), (0,)), ((), ()))
    gates = (
        jnp.dot(xt, wih_ref[...], preferred_element_type=jnp.float32)
        + lax.dot_general(ht_scr[...], whh_ref[...], dn0,
                          preferred_element_type=jnp.float32)
        + b_ref[...]
    )
    i = jax.nn.sigmoid(gates[:, 0 * EMBED : 1 * EMBED])
    f = jax.nn.sigmoid(gates[:, 1 * EMBED : 2 * EMBED])
    g = jnp.tanh(gates[:, 2 * EMBED : 3 * EMBED])
    o = jax.nn.sigmoid(gates[:, 3 * EMBED : 4 * EMBED])
    c = f * c_scr[...] + i * g
    h = o * jnp.tanh(c)
    c_scr[...] = c
    eye = jnp.eye(EMBED, dtype=jnp.float32)
    h_t = lax.dot_general(
        eye, h, (((1,), (1,)), ((), ())),
        precision=lax.Precision.HIGHEST,
        preferred_element_type=jnp.float32,
    )
    ht_scr[...] = h_t
    out_ref[...] = h_t


def _lstm(x2d, wih_t, whh_t, b):
    G = 4 * EMBED
    return pl.pallas_call(
        _lstm_body,
        grid=(SEQ,),
        in_specs=[
            pl.BlockSpec((BATCH // 2, 128), lambda t: (t, 0)),
            pl.BlockSpec((EMBED, G), lambda t: (0, 0)),
            pl.BlockSpec((EMBED, G), lambda t: (0, 0)),
            pl.BlockSpec((1, G), lambda t: (0, 0)),
        ],
        out_specs=pl.BlockSpec((EMBED, BATCH), lambda t: (0, t)),
        out_shape=jax.ShapeDtypeStruct((EMBED, NTOK), jnp.float32),
        scratch_shapes=[
            pltpu.VMEM((EMBED, BATCH), jnp.float32),
            pltpu.VMEM((BATCH, EMBED), jnp.float32),
        ],
    )(x2d, wih_t, whh_t, b)


def kernel(samples, text, targets, in_embed, out_embed, W_ih, W_hh, b_ih, b_hh):
    E = in_embed.shape[1]
    sample_size = samples.shape[-1]

    txt_idx = text.reshape(-1).astype(jnp.int32)
    tgt_idx = targets.reshape(-1).astype(jnp.int32)
    samp_idx = jnp.transpose(samples, (2, 0, 1)).reshape(-1).astype(jnp.int32)

    ja, jb = 8, 16
    samp_emb_a = _sc_gather_paired_db(out_embed, samp_idx[: ja * NTOK], W=128, K=2)
    samp_emb_b = _sc_gather_paired_db(
        out_embed, samp_idx[ja * NTOK : jb * NTOK], W=128, K=2
    )
    samp_emb_c = _sc_gather_paired_db(out_embed, samp_idx[jb * NTOK :], W=128, K=1)
    txt_emb = _sc_gather_paired(in_embed, txt_idx, W=64, K=1)
    rnn_t = _lstm(
        txt_emb.reshape(NTOK // 2, 128),
        W_ih.T,
        W_hh.T,
        (b_ih + b_hh).reshape(1, -1),
    )
    tgt_emb = _sc_gather_paired(out_embed, tgt_idx, W=64, K=1)
    tgt_t = _tc_transpose(tgt_emb, 1, NTOK)
    samp_t = _tc_transpose_into(samp_emb_a, None, 0, ja, sample_size)
    samp_t = _tc_transpose_into(samp_emb_b, samp_t, ja, jb - ja, sample_size)
    samp_t = _tc_transpose_into(
        samp_emb_c, samp_t, jb, sample_size - jb, sample_size
    )

    return (
        jnp.transpose(samp_t, (2, 0, 1)),
        jnp.transpose(rnn_t, (1, 0))[:, :, None],
        jnp.transpose(tgt_t, (2, 0, 1)),
    )


# speedup vs baseline: 1.0532x; 1.0532x over previous
import functools

import jax
import jax.numpy as jnp
from jax import lax
from jax.experimental import pallas as pl
from jax.experimental.pallas import tpu as pltpu
from jax.experimental.pallas import tpu_sc as plsc

NC = 2
NS = 16
NW = NC * NS

EMBED = 64
SEQ = 50
BATCH = 1024
SAMPLE = 20
NTOK = SEQ * BATCH
HALF = 512


def _sc_gather_paired(table, idx, W, K):
    N = idx.shape[0]
    D = table.shape[1]
    n_per_w = N // NW
    R = n_per_w // W
    n_chunks = R // K
    F = K * W
    assert N == NW * R * W and R == n_chunks * K
    assert HALF % F == 0 and N % 1024 == 0

    mesh = plsc.VectorSubcoreMesh(core_axis_name="c", subcore_axis_name="s")

    @functools.partial(
        pl.kernel,
        mesh=mesh,
        out_type=jax.ShapeDtypeStruct((N // 1024, HALF, 2, D), jnp.float32),
        compiler_params=pltpu.CompilerParams(use_tc_tiling_on_sc=False),
        scratch_types=[
            pltpu.VMEM((R, W), jnp.int32),
            pltpu.VMEM((F, D), jnp.float32),
            pltpu.SemaphoreType.DMA,
        ],
    )
    def gather_kernel(table_hbm, idx_hbm, out_hbm, idx_v, rows_v, sem):
        wid = lax.axis_index("s") * NC + lax.axis_index("c")
        base = wid * n_per_w
        pltpu.sync_copy(idx_hbm.at[wid], idx_v)

        def chunk_body(i, carry):
            cps = []
            for j in range(K):
                cps.append(
                    pltpu.async_copy(
                        table_hbm.at[idx_v.at[i * K + j]],
                        rows_v.at[pl.ds(j * W, W)],
                        sem,
                    )
                )
            for cp in cps:
                cp.wait()
            a = base + i * F
            c = a // 1024
            m = a % 1024
            h = m // HALF
            rr = m % HALF
            pltpu.sync_copy(rows_v, out_hbm.at[c, pl.ds(rr, F), h])
            return carry

        lax.fori_loop(0, n_chunks, chunk_body, 0)

    return gather_kernel(table, idx.reshape(NW, R, W))


def _sc_gather_paired_db(table, idx, W, K):
    N = idx.shape[0]
    D = table.shape[1]
    n_per_w = N // NW
    R = n_per_w // W
    n_chunks = R // K
    F = K * W
    assert N == NW * R * W and R == n_chunks * K and n_chunks % 2 == 0
    assert HALF % F == 0 and N % 1024 == 0
    n2 = n_chunks // 2

    mesh = plsc.VectorSubcoreMesh(core_axis_name="c", subcore_axis_name="s")

    @functools.partial(
        pl.kernel,
        mesh=mesh,
        out_type=jax.ShapeDtypeStruct((N // 1024, HALF, 2, D), jnp.float32),
        compiler_params=pltpu.CompilerParams(use_tc_tiling_on_sc=False),
        scratch_types=[
            pltpu.VMEM((R, W), jnp.int32),
            pltpu.VMEM((F, D), jnp.float32),
            pltpu.VMEM((F, D), jnp.float32),
            pltpu.SemaphoreType.DMA,
            pltpu.SemaphoreType.DMA,
            pltpu.SemaphoreType.DMA,
            pltpu.SemaphoreType.DMA,
        ],
    )
    def gather_kernel(table_hbm, idx_hbm, out_hbm, idx_v, rows0, rows1,
                      semg0, semg1, semf0, semf1):
        wid = lax.axis_index("s") * NC + lax.axis_index("c")
        base = wid * n_per_w
        pltpu.sync_copy(idx_hbm.at[wid], idx_v)

        def fire(i, buf, sem):
            for j in range(K):
                pltpu.async_copy(
                    table_hbm.at[idx_v.at[i * K + j]],
                    buf.at[pl.ds(j * W, W)],
                    sem,
                )

        def wait_gathers(buf, sem):
            for j in range(K):
                pltpu.make_async_copy(
                    table_hbm.at[idx_v.at[0]], buf.at[pl.ds(j * W, W)], sem
                ).wait()

        def dst(i):
            a = base + i * F
            return out_hbm.at[a // 1024, pl.ds((a % 1024) % HALF, F),
                              (a % 1024) // HALF]

        fire(0, rows0, semg0)

        def body(p, carry):
            i0 = p * 2
            wait_gathers(rows0, semg0)

            @pl.when(p > 0)
            def _():
                pltpu.make_async_copy(rows1, dst(i0 - 1), semf1).wait()

            fire(i0 + 1, rows1, semg1)
            pltpu.async_copy(rows0, dst(i0), semf0)
            wait_gathers(rows1, semg1)
            pltpu.make_async_copy(rows0, dst(i0), semf0).wait()

            @pl.when(p < n2 - 1)
            def _():
                fire(i0 + 2, rows0, semg0)

            pltpu.async_copy(rows1, dst(i0 + 1), semf1)
            return carry

        lax.fori_loop(0, n2, body, 0)
        pltpu.make_async_copy(rows1, dst(n_chunks - 1), semf1).wait()

    return gather_kernel(table, idx.reshape(NW, R, W))


_TB = 10240


def _transpose_body(x_ref, o_ref):
    x = x_ref[0]
    eye = jnp.eye(128, dtype=jnp.float32)
    dn = (((1,), (1,)), ((), ()))
    y = lax.dot_general(eye, x, dn,
                        precision=lax.Precision.HIGHEST,
                        preferred_element_type=jnp.float32)
    for g in range(_TB // 1024):
        o_ref[0, :, g * 1024 : g * 1024 + HALF] = y[0:EMBED, g * HALF : (g + 1) * HALF]
        o_ref[0, :, g * 1024 + HALF : (g + 1) * 1024] = y[EMBED:128, g * HALF : (g + 1) * HALF]


def _tc_transpose(x_paired, nj, T):
    x3 = x_paired.reshape(nj, T // 2, 128)
    n_c = T // _TB
    return pl.pallas_call(
        _transpose_body,
        grid=(nj, n_c),
        in_specs=[pl.BlockSpec((1, _TB // 2, 128), lambda j, c: (j, c, 0))],
        out_specs=pl.BlockSpec((1, EMBED, _TB), lambda j, c: (j, 0, c)),
        out_shape=jax.ShapeDtypeStruct((nj, EMBED, T), jnp.float32),
    )(x3)


def _tc_transpose_into(x_paired, prev, j0, nj_chunk, nj_total):
    x3 = x_paired.reshape(nj_chunk, NTOK // 2, 128)
    n_c = NTOK // _TB
    out_sds = jax.ShapeDtypeStruct((nj_total, EMBED, NTOK), jnp.float32)
    if prev is None:
        return pl.pallas_call(
            _transpose_body,
            grid=(nj_chunk, n_c),
            in_specs=[pl.BlockSpec((1, _TB // 2, 128), lambda j, c: (j, c, 0))],
            out_specs=pl.BlockSpec((1, EMBED, _TB), lambda j, c: (j + j0, 0, c)),
            out_shape=out_sds,
        )(x3)

    def body(x_ref, prev_ref, o_ref):
        _transpose_body(x_ref, o_ref)

    return pl.pallas_call(
        body,
        grid=(nj_chunk, n_c),
        in_specs=[
            pl.BlockSpec((1, _TB // 2, 128), lambda j, c: (j, c, 0)),
            pl.BlockSpec(memory_space=pl.ANY),
        ],
        out_specs=pl.BlockSpec((1, EMBED, _TB), lambda j, c: (j + j0, 0, c)),
        out_shape=out_sds,
        input_output_aliases={1: 0},
    )(x3, prev)


def _lstm_body(x_ref, wih_ref, whh_ref, b_ref, out_ref, ht_scr, c_scr):
    t = pl.program_id(0)

    @pl.when(t == 0)
    def _init():
        ht_scr[...] = jnp.zeros_like(ht_scr)
        c_scr[...] = jnp.zeros_like(c_scr)

    x2 = x_ref[...]
    xt = jnp.concatenate([x2[:, 0:EMBED], x2[:, EMBED:128]], axis=0)
    dn0 = (((0,), (0,)), ((), ()))
    gates = (
        jnp.dot(xt, wih_ref[...], preferred_element_type=jnp.float32)
        + lax.dot_general(ht_scr[...], whh_ref[...], dn0,
                          preferred_element_type=jnp.float32)
        + b_ref[...]
    )
    i = jax.nn.sigmoid(gates[:, 0 * EMBED : 1 * EMBED])
    f = jax.nn.sigmoid(gates[:, 1 * EMBED : 2 * EMBED])
    g = jnp.tanh(gates[:, 2 * EMBED : 3 * EMBED])
    o = jax.nn.sigmoid(gates[:, 3 * EMBED : 4 * EMBED])
    c = f * c_scr[...] + i * g
    h = o * jnp.tanh(c)
    c_scr[...] = c
    eye = jnp.eye(EMBED, dtype=jnp.float32)
    h_t = lax.dot_general(
        eye, h, (((1,), (1,)), ((), ())),
        precision=lax.Precision.HIGHEST,
        preferred_element_type=jnp.float32,
    )
    ht_scr[...] = h_t
    out_ref[...] = h_t


def _lstm(x2d, wih_t, whh_t, b):
    G = 4 * EMBED
    return pl.pallas_call(
        _lstm_body,
        grid=(SEQ,),
        in_specs=[
            pl.BlockSpec((BATCH // 2, 128), lambda t: (t, 0)),
            pl.BlockSpec((EMBED, G), lambda t: (0, 0)),
            pl.BlockSpec((EMBED, G), lambda t: (0, 0)),
            pl.BlockSpec((1, G), lambda t: (0, 0)),
        ],
        out_specs=pl.BlockSpec((EMBED, BATCH), lambda t: (0, t)),
        out_shape=jax.ShapeDtypeStruct((EMBED, NTOK), jnp.float32),
        scratch_shapes=[
            pltpu.VMEM((EMBED, BATCH), jnp.float32),
            pltpu.VMEM((BATCH, EMBED), jnp.float32),
        ],
    )(x2d, wih_t, whh_t, b)


def kernel(samples, text, targets, in_embed, out_embed, W_ih, W_hh, b_ih, b_hh):
    E = in_embed.shape[1]
    sample_size = samples.shape[-1]

    txt_idx = text.reshape(-1).astype(jnp.int32)
    tgt_idx = targets.reshape(-1).astype(jnp.int32)
    samp_idx = jnp.transpose(samples, (2, 0, 1)).reshape(-1).astype(jnp.int32)

    ja, jb = 8, 16
    samp_emb_a = _sc_gather_paired(out_embed, samp_idx[: ja * NTOK], W=128, K=2)
    samp_emb_b = _sc_gather_paired(
        out_embed, samp_idx[ja * NTOK : jb * NTOK], W=128, K=2
    )
    samp_emb_c = _sc_gather_paired(out_embed, samp_idx[jb * NTOK :], W=128, K=2)
    txt_emb = _sc_gather_paired(in_embed, txt_idx, W=64, K=1)
    rnn_t = _lstm(
        txt_emb.reshape(NTOK // 2, 128),
        W_ih.T,
        W_hh.T,
        (b_ih + b_hh).reshape(1, -1),
    )
    tgt_emb = _sc_gather_paired(out_embed, tgt_idx, W=64, K=1)
    tgt_t = _tc_transpose(tgt_emb, 1, NTOK)
    samp_t = _tc_transpose_into(samp_emb_a, None, 0, ja, sample_size)
    samp_t = _tc_transpose_into(samp_emb_b, samp_t, ja, jb - ja, sample_size)
    samp_t = _tc_transpose_into(
        samp_emb_c, samp_t, jb, sample_size - jb, sample_size
    )

    return (
        jnp.transpose(samp_t, (2, 0, 1)),
        jnp.transpose(rnn_t, (1, 0))[:, :, None],
        jnp.transpose(tgt_t, (2, 0, 1)),
    )


# speedup vs baseline: 1.1308x; 1.0737x over previous
import functools

import jax
import jax.numpy as jnp
from jax import lax
from jax.experimental import pallas as pl
from jax.experimental.pallas import tpu as pltpu
from jax.experimental.pallas import tpu_sc as plsc

NC = 2
NS = 16
NW = NC * NS

EMBED = 64
SEQ = 50
BATCH = 1024
SAMPLE = 20
NTOK = SEQ * BATCH
HALF = 512


def _sc_gather_paired(table, idx, W, K):
    N = idx.shape[0]
    D = table.shape[1]
    n_per_w = N // NW
    R = n_per_w // W
    n_chunks = R // K
    F = K * W
    assert N == NW * R * W and R == n_chunks * K
    assert HALF % F == 0 and N % 1024 == 0

    mesh = plsc.VectorSubcoreMesh(core_axis_name="c", subcore_axis_name="s")

    @functools.partial(
        pl.kernel,
        mesh=mesh,
        out_type=jax.ShapeDtypeStruct((N // 1024, HALF, 2, D), jnp.float32),
        compiler_params=pltpu.CompilerParams(use_tc_tiling_on_sc=False),
        scratch_types=[
            pltpu.VMEM((R, W), jnp.int32),
            pltpu.VMEM((F, D), jnp.float32),
            pltpu.SemaphoreType.DMA,
        ],
    )
    def gather_kernel(table_hbm, idx_hbm, out_hbm, idx_v, rows_v, sem):
        wid = lax.axis_index("s") * NC + lax.axis_index("c")
        base = wid * n_per_w
        pltpu.sync_copy(idx_hbm.at[wid], idx_v)

        def chunk_body(i, carry):
            cps = []
            for j in range(K):
                cps.append(
                    pltpu.async_copy(
                        table_hbm.at[idx_v.at[i * K + j]],
                        rows_v.at[pl.ds(j * W, W)],
                        sem,
                    )
                )
            for cp in cps:
                cp.wait()
            a = base + i * F
            c = a // 1024
            m = a % 1024
            h = m // HALF
            rr = m % HALF
            pltpu.sync_copy(rows_v, out_hbm.at[c, pl.ds(rr, F), h])
            return carry

        lax.fori_loop(0, n_chunks, chunk_body, 0)

    return gather_kernel(table, idx.reshape(NW, R, W))


_TB = 25600


def _transpose_body(x_ref, o_ref):
    x = x_ref[0]
    eye = jnp.eye(128, dtype=jnp.float32)
    dn = (((1,), (1,)), ((), ()))
    y = lax.dot_general(eye, x, dn,
                        precision=lax.Precision.HIGHEST,
                        preferred_element_type=jnp.float32)
    for g in range(_TB // 1024):
        o_ref[0, :, g * 1024 : g * 1024 + HALF] = y[0:EMBED, g * HALF : (g + 1) * HALF]
        o_ref[0, :, g * 1024 + HALF : (g + 1) * 1024] = y[EMBED:128, g * HALF : (g + 1) * HALF]


def _tc_transpose(x_paired, nj, T):
    x3 = x_paired.reshape(nj, T // 2, 128)
    n_c = T // _TB
    return pl.pallas_call(
        _transpose_body,
        grid=(nj, n_c),
        in_specs=[pl.BlockSpec((1, _TB // 2, 128), lambda j, c: (j, c, 0))],
        out_specs=pl.BlockSpec((1, EMBED, _TB), lambda j, c: (j, 0, c)),
        out_shape=jax.ShapeDtypeStruct((nj, EMBED, T), jnp.float32),
    )(x3)


def _tc_transpose_into(x_paired, prev, j0, nj_chunk, nj_total):
    x3 = x_paired.reshape(nj_chunk, NTOK // 2, 128)
    n_c = NTOK // _TB
    out_sds = jax.ShapeDtypeStruct((nj_total, EMBED, NTOK), jnp.float32)
    if prev is None:
        return pl.pallas_call(
            _transpose_body,
            grid=(nj_chunk, n_c),
            in_specs=[pl.BlockSpec((1, _TB // 2, 128), lambda j, c: (j, c, 0))],
            out_specs=pl.BlockSpec((1, EMBED, _TB), lambda j, c: (j + j0, 0, c)),
            out_shape=out_sds,
        )(x3)

    def body(x_ref, prev_ref, o_ref):
        _transpose_body(x_ref, o_ref)

    return pl.pallas_call(
        body,
        grid=(nj_chunk, n_c),
        in_specs=[
            pl.BlockSpec((1, _TB // 2, 128), lambda j, c: (j, c, 0)),
            pl.BlockSpec(memory_space=pl.ANY),
        ],
        out_specs=pl.BlockSpec((1, EMBED, _TB), lambda j, c: (j + j0, 0, c)),
        out_shape=out_sds,
        input_output_aliases={1: 0},
    )(x3, prev)


def _lstm_body(x_ref, wih_ref, whh_ref, b_ref, out_ref, ht_scr, c_scr):
    t = pl.program_id(0)

    @pl.when(t == 0)
    def _init():
        ht_scr[...] = jnp.zeros_like(ht_scr)
        c_scr[...] = jnp.zeros_like(c_scr)

    x2 = x_ref[...]
    xt = jnp.concatenate([x2[:, 0:EMBED], x2[:, EMBED:128]], axis=0)
    dn0 = (((0,), (0,)), ((), ()))
    gates = (
        jnp.dot(xt, wih_ref[...], preferred_element_type=jnp.float32)
        + lax.dot_general(ht_scr[...], whh_ref[...], dn0,
                          preferred_element_type=jnp.float32)
        + b_ref[...]
    )
    i = jax.nn.sigmoid(gates[:, 0 * EMBED : 1 * EMBED])
    f = jax.nn.sigmoid(gates[:, 1 * EMBED : 2 * EMBED])
    g = jnp.tanh(gates[:, 2 * EMBED : 3 * EMBED])
    o = jax.nn.sigmoid(gates[:, 3 * EMBED : 4 * EMBED])
    c = f * c_scr[...] + i * g
    h = o * jnp.tanh(c)
    c_scr[...] = c
    eye = jnp.eye(EMBED, dtype=jnp.float32)
    h_t = lax.dot_general(
        eye, h, (((1,), (1,)), ((), ())),
        precision=lax.Precision.HIGHEST,
        preferred_element_type=jnp.float32,
    )
    ht_scr[...] = h_t
    out_ref[...] = h_t


def _lstm(x2d, wih_t, whh_t, b):
    G = 4 * EMBED
    return pl.pallas_call(
        _lstm_body,
        grid=(SEQ,),
        in_specs=[
            pl.BlockSpec((BATCH // 2, 128), lambda t: (t, 0)),
            pl.BlockSpec((EMBED, G), lambda t: (0, 0)),
            pl.BlockSpec((EMBED, G), lambda t: (0, 0)),
            pl.BlockSpec((1, G), lambda t: (0, 0)),
        ],
        out_specs=pl.BlockSpec((EMBED, BATCH), lambda t: (0, t)),
        out_shape=jax.ShapeDtypeStruct((EMBED, NTOK), jnp.float32),
        scratch_shapes=[
            pltpu.VMEM((EMBED, BATCH), jnp.float32),
            pltpu.VMEM((BATCH, EMBED), jnp.float32),
        ],
    )(x2d, wih_t, whh_t, b)


def kernel(samples, text, targets, in_embed, out_embed, W_ih, W_hh, b_ih, b_hh):
    E = in_embed.shape[1]
    sample_size = samples.shape[-1]

    txt_idx = text.reshape(-1).astype(jnp.int32)
    tgt_idx = targets.reshape(-1).astype(jnp.int32)
    samp_idx = jnp.transpose(samples, (2, 0, 1)).reshape(-1).astype(jnp.int32)

    ja, jb = 8, 16
    samp_emb_a = _sc_gather_paired(out_embed, samp_idx[: ja * NTOK], W=128, K=4)
    samp_emb_b = _sc_gather_paired(
        out_embed, samp_idx[ja * NTOK : jb * NTOK], W=128, K=4
    )
    samp_emb_c = _sc_gather_paired(out_embed, samp_idx[jb * NTOK :], W=128, K=2)
    txt_emb = _sc_gather_paired(in_embed, txt_idx, W=64, K=1)
    rnn_t = _lstm(
        txt_emb.reshape(NTOK // 2, 128),
        W_ih.T,
        W_hh.T,
        (b_ih + b_hh).reshape(1, -1),
    )
    tgt_emb = _sc_gather_paired(out_embed, tgt_idx, W=64, K=1)
    tgt_t = _tc_transpose(tgt_emb, 1, NTOK)
    samp_t = _tc_transpose_into(samp_emb_a, None, 0, ja, sample_size)
    samp_t = _tc_transpose_into(samp_emb_b, samp_t, ja, jb - ja, sample_size)
    samp_t = _tc_transpose_into(
        samp_emb_c, samp_t, jb, sample_size - jb, sample_size
    )

    return (
        jnp.transpose(samp_t, (2, 0, 1)),
        jnp.transpose(rnn_t, (1, 0))[:, :, None],
        jnp.transpose(tgt_t, (2, 0, 1)),
    )
